# baseline (speedup 1.0000x reference)
import functools

import jax
import jax.numpy as jnp
from jax import lax
from jax.experimental import pallas as pl
from jax.experimental.pallas import tpu as pltpu
from jax.experimental.pallas import tpu_sc as plsc

N_LOC = 10000
N_EXP = 10000
E = 320000
D = 128
H = 128
OUT = 128

NC = 2
NS = 16
NW = NC * NS
K = 64
NB = 4
NI = 8
NCH = 160
EPW = NCH * K
E_PAD = NW * EPW
IDX_ROWS = NW * NCH + NI
N_ACC = 10240
RPS = N_ACC // NS
G16 = K // 16

_sc_mesh = plsc.VectorSubcoreMesh(core_axis_name="c", subcore_axis_name="s")


@functools.partial(
    pl.kernel,
    mesh=_sc_mesh,
    compiler_params=pltpu.CompilerParams(needs_layout_passes=False),
    out_type=(
        jax.ShapeDtypeStruct((NC, N_ACC, D), jnp.float32),
        jax.ShapeDtypeStruct((NC, N_ACC), jnp.float32),
        jax.ShapeDtypeStruct((NC, NS, N_ACC), jnp.float32),
    ),
    scratch_types=[
        [pltpu.VMEM((2 * K,), jnp.int32) for _ in range(NI)],
        [pltpu.VMEM((K, D), jnp.float32) for _ in range(NB)],
        pltpu.VMEM((K,), jnp.int32),
        pltpu.VMEM((N_ACC,), jnp.float32),
        pltpu.VMEM((RPS,), jnp.float32),
        pltpu.VMEM((RPS,), jnp.float32),
        pltpu.VMEM_SHARED((N_ACC, D), jnp.float32),
        [pltpu.SemaphoreType.DMA for _ in range(NI)],
        [pltpu.SemaphoreType.DMA for _ in range(NB)],
    ],
)
def _sc_segment_sum(x_loc, idx2d, zrows, out_feat, out_cnt, out_hist,
                    islots, bufs, dst_v, hist_v, cred_v, tmp_v,
                    acc_sh, isems, rsems):
    c = lax.axis_index("c")
    s = lax.axis_index("s")
    wid = s * NC + c
    row0 = wid * NCH

    def idx_issue(m, slot):
        pltpu.async_copy(idx2d.at[row0 + m], islots[slot], isems[slot])

    def idx_wait(m, slot):
        pltpu.make_async_copy(idx2d.at[row0 + m], islots[slot],
                              isems[slot]).wait()

    def gather_issue(m, slot, b):
        pltpu.async_copy(x_loc.at[islots[slot].at[pl.ds(0, K)]], bufs[b],
                         rsems[b])

    def gather_wait(m, slot, b):
        pltpu.make_async_copy(x_loc.at[islots[slot].at[pl.ds(0, K)]], bufs[b],
                              rsems[b]).wait()

    zeros16 = jnp.zeros((16,), jnp.float32)

    def zh(k, carry):
        hist_v[pl.ds(k * 16, 16)] = zeros16
        return carry

    lax.fori_loop(0, N_ACC // 16, zh, 0)
    pltpu.sync_copy(zrows.at[pl.ds(s * RPS, RPS)],
                    acc_sh.at[pl.ds(s * RPS, RPS)])
    plsc.subcore_barrier()

    ones16 = jnp.ones((16,), jnp.float32)

    for m in range(NI - 1):
        idx_issue(m, m)
    for m in range(NB - 1):
        idx_wait(m, m)
        gather_issue(m, m, m)

    def outer(i, carry):
        for b in range(NI):
            ch = i * NI + b
            idx_issue(ch + NI - 1, (b + NI - 1) % NI)
            idx_wait(ch + NB - 1, (b + NB - 1) % NI)
            gather_issue(ch + NB - 1, (b + NB - 1) % NI, (b + NB - 1) % NB)
            gather_wait(ch, b % NI, b % NB)
            for g in range(G16):
                v = islots[b % NI][pl.ds(K + g * 16, 16)]
                dst_v[pl.ds(g * 16, 16)] = v
                plsc.addupdate_scatter(hist_v, [v], ones16)
            pltpu.sync_copy(bufs[b % NB], acc_sh.at[dst_v], add=True)
        return carry

    lax.fori_loop(0, NCH // NI, outer, 0)

    for m in range(NB - 1):
        gather_wait(NCH + m, (NCH + m) % NI, (NCH + m) % NB)
    for m in range(NB - 1, NI - 1):
        idx_wait(NCH + m, (NCH + m) % NI)

    pltpu.sync_copy(hist_v, out_hist.at[c].at[s])
    plsc.subcore_barrier()

    def czero(g, carry):
        cred_v[pl.ds(g * 16, 16)] = zeros16
        return carry

    lax.fori_loop(0, RPS // 16, czero, 0)
    for j in range(NS):
        pltpu.sync_copy(out_hist.at[c].at[j].at[pl.ds(s * RPS, RPS)], tmp_v)

        def cadd(g, carry):
            sl = pl.ds(g * 16, 16)
            cred_v[sl] = cred_v[sl] + tmp_v[sl]
            return carry

        lax.fori_loop(0, RPS // 16, cadd, 0)

    pltpu.sync_copy(acc_sh.at[pl.ds(s * RPS, RPS)],
                    out_feat.at[c].at[pl.ds(s * RPS, RPS)])
    pltpu.sync_copy(cred_v, out_cnt.at[c].at[pl.ds(s * RPS, RPS)])


def _tc_body(p0, p1, c0, c1, xe, wl, wr, wo, bl, bo, o):
    cnt = jnp.maximum(c0[0] + c1[0], 1.0)
    sacc = p0[0] + p1[0]
    mean = sacc / cnt
    h = jnp.dot(mean, wl[...], preferred_element_type=jnp.float32)
    h = h + jnp.dot(xe[...], wr[...], preferred_element_type=jnp.float32)
    h = jnp.maximum(h + bl[...], 0.0)
    o[...] = jnp.dot(h, wo[...], preferred_element_type=jnp.float32) + bo[...]


BT = 1024


def _tc_stage(parts, cnts, x_expert, wlT, wrT, woT, bl, bo):
    grid = (-(-N_EXP // BT),)
    return pl.pallas_call(
        _tc_body,
        grid=grid,
        in_specs=[
            pl.BlockSpec((1, BT, D), lambda i: (0, i, 0)),
            pl.BlockSpec((1, BT, D), lambda i: (1, i, 0)),
            pl.BlockSpec((1, BT, 1), lambda i: (0, i, 0)),
            pl.BlockSpec((1, BT, 1), lambda i: (1, i, 0)),
            pl.BlockSpec((BT, D), lambda i: (i, 0)),
            pl.BlockSpec((D, H), lambda i: (0, 0)),
            pl.BlockSpec((D, H), lambda i: (0, 0)),
            pl.BlockSpec((H, OUT), lambda i: (0, 0)),
            pl.BlockSpec((1, H), lambda i: (0, 0)),
            pl.BlockSpec((1, OUT), lambda i: (0, 0)),
        ],
        out_specs=pl.BlockSpec((BT, OUT), lambda i: (i, 0)),
        out_shape=jax.ShapeDtypeStruct((N_EXP, OUT), jnp.float32),
    )(parts, parts, cnts, cnts, x_expert, wlT, wrT, woT, bl, bo)


def kernel(x_loc, x_expert, edge_index, W_l, b_l, W_r, W_lin, b_lin):
    src = edge_index[0]
    dst = edge_index[1]
    pad = IDX_ROWS * K - E
    src_p = jnp.concatenate([src, jnp.zeros((pad,), jnp.int32)])
    dst_p = jnp.concatenate([dst, jnp.full((pad,), N_EXP, jnp.int32)])
    idx2d = jnp.concatenate([src_p.reshape(IDX_ROWS, K),
                             dst_p.reshape(IDX_ROWS, K)], axis=1)
    zrows = jnp.zeros((N_ACC, D), jnp.float32)

    parts, cnts, _ = _sc_segment_sum(x_loc, idx2d, zrows)
    return _tc_stage(parts, cnts.reshape(NC, N_ACC, 1), x_expert,
                     W_l.T, W_r.T, W_lin.T, b_l[None, :], b_lin[None, :])


# speedup vs baseline: 1.0052x; 1.0052x over previous
import functools

import jax
import jax.numpy as jnp
from jax import lax
from jax.experimental import pallas as pl
from jax.experimental.pallas import tpu as pltpu
from jax.experimental.pallas import tpu_sc as plsc

N_LOC = 10000
N_EXP = 10000
E = 320000
D = 128
H = 128
OUT = 128

NC = 2
NS = 16
NW = NC * NS
K = 64
NB = 4
GA = 2
NI = 8
NCH = 160
EPW = NCH * K
E_PAD = NW * EPW
IDX_ROWS = NW * NCH + NI
N_ACC = 10240
RPS = N_ACC // NS
G16 = K // 16

_sc_mesh = plsc.VectorSubcoreMesh(core_axis_name="c", subcore_axis_name="s")


@functools.partial(
    pl.kernel,
    mesh=_sc_mesh,
    compiler_params=pltpu.CompilerParams(needs_layout_passes=False),
    out_type=(
        jax.ShapeDtypeStruct((NC, N_ACC, D), jnp.float32),
        jax.ShapeDtypeStruct((NC, N_ACC), jnp.float32),
        jax.ShapeDtypeStruct((NC, NS, N_ACC), jnp.float32),
    ),
    scratch_types=[
        [pltpu.VMEM((2 * K,), jnp.int32) for _ in range(NI)],
        [pltpu.VMEM((K, D), jnp.float32) for _ in range(NB)],
        [pltpu.VMEM((K,), jnp.int32) for _ in range(NB)],
        pltpu.VMEM((N_ACC,), jnp.float32),
        pltpu.VMEM((RPS,), jnp.float32),
        pltpu.VMEM((RPS,), jnp.float32),
        pltpu.VMEM_SHARED((N_ACC, D), jnp.float32),
        [pltpu.SemaphoreType.DMA for _ in range(NI)],
        [pltpu.SemaphoreType.DMA for _ in range(NB)],
        [pltpu.SemaphoreType.DMA for _ in range(NB)],
    ],
)
def _sc_segment_sum(x_loc, idx2d, zrows, out_feat, out_cnt, out_hist,
                    islots, bufs, dsts, hist_v, cred_v, tmp_v,
                    acc_sh, isems, gsems, ssems):
    c = lax.axis_index("c")
    s = lax.axis_index("s")
    wid = s * NC + c
    row0 = wid * NCH

    def idx_issue(m, slot):
        pltpu.async_copy(idx2d.at[row0 + m], islots[slot], isems[slot])

    def idx_wait(m, slot):
        pltpu.make_async_copy(idx2d.at[row0 + m], islots[slot],
                              isems[slot]).wait()

    def gather_issue(slot, b):
        pltpu.async_copy(x_loc.at[islots[slot].at[pl.ds(0, K)]], bufs[b],
                         gsems[b])

    def gather_wait(slot, b):
        pltpu.make_async_copy(x_loc.at[islots[slot].at[pl.ds(0, K)]], bufs[b],
                              gsems[b]).wait()

    def scatter_issue(b):
        pltpu.async_copy(bufs[b], acc_sh.at[dsts[b]], ssems[b], add=True)

    def scatter_wait(b):
        pltpu.make_async_copy(bufs[b], acc_sh.at[dsts[b]], ssems[b]).wait()

    zeros16 = jnp.zeros((16,), jnp.float32)

    def zh(k, carry):
        hist_v[pl.ds(k * 16, 16)] = zeros16
        return carry

    lax.fori_loop(0, N_ACC // 16, zh, 0)
    pltpu.sync_copy(zrows.at[pl.ds(s * RPS, RPS)],
                    acc_sh.at[pl.ds(s * RPS, RPS)])
    plsc.subcore_barrier()

    ones16 = jnp.ones((16,), jnp.float32)

    for m in range(NI - 1):
        idx_issue(m, m)
    for m in range(GA):
        idx_wait(m, m)
        gather_issue(m, m)

    def outer(i, carry):
        for b in range(NI):
            ch = i * NI + b
            idx_issue(ch + NI - 1, (b + NI - 1) % NI)
            if b < NB - GA:
                @pl.when(i >= 1)
                def _drain():
                    scatter_wait((b + GA) % NB)
            else:
                scatter_wait((b + GA) % NB)
            idx_wait(ch + GA, (b + GA) % NI)
            gather_issue((b + GA) % NI, (b + GA) % NB)
            gather_wait(b % NI, b % NB)
            for g in range(G16):
                v = islots[b % NI][pl.ds(K + g * 16, 16)]
                dsts[b % NB][pl.ds(g * 16, 16)] = v
                plsc.addupdate_scatter(hist_v, [v], ones16)
            scatter_issue(b % NB)
        return carry

    lax.fori_loop(0, NCH // NI, outer, 0)

    for m in range(GA):
        gather_wait((NCH + m) % NI, (NCH + m) % NB)
    for m in range(GA, NI - 1):
        idx_wait(NCH + m, (NCH + m) % NI)
    for m in range(NB - GA):
        scatter_wait((NCH - (NB - GA) + m) % NB)

    pltpu.sync_copy(hist_v, out_hist.at[c].at[s])
    plsc.subcore_barrier()

    def czero(g, carry):
        cred_v[pl.ds(g * 16, 16)] = zeros16
        return carry

    lax.fori_loop(0, RPS // 16, czero, 0)
    for j in range(NS):
        pltpu.sync_copy(out_hist.at[c].at[j].at[pl.ds(s * RPS, RPS)], tmp_v)

        def cadd(g, carry):
            sl = pl.ds(g * 16, 16)
            cred_v[sl] = cred_v[sl] + tmp_v[sl]
            return carry

        lax.fori_loop(0, RPS // 16, cadd, 0)

    pltpu.sync_copy(acc_sh.at[pl.ds(s * RPS, RPS)],
                    out_feat.at[c].at[pl.ds(s * RPS, RPS)])
    pltpu.sync_copy(cred_v, out_cnt.at[c].at[pl.ds(s * RPS, RPS)])


def _tc_body(p0, p1, c0, c1, xe, wl, wr, wo, bl, bo, o):
    cnt = jnp.maximum(c0[0] + c1[0], 1.0)
    sacc = p0[0] + p1[0]
    mean = sacc / cnt
    h = jnp.dot(mean, wl[...], preferred_element_type=jnp.float32)
    h = h + jnp.dot(xe[...], wr[...], preferred_element_type=jnp.float32)
    h = jnp.maximum(h + bl[...], 0.0)
    o[...] = jnp.dot(h, wo[...], preferred_element_type=jnp.float32) + bo[...]


BT = 1024


def _tc_stage(parts, cnts, x_expert, wlT, wrT, woT, bl, bo):
    grid = (-(-N_EXP // BT),)
    return pl.pallas_call(
        _tc_body,
        grid=grid,
        in_specs=[
            pl.BlockSpec((1, BT, D), lambda i: (0, i, 0)),
            pl.BlockSpec((1, BT, D), lambda i: (1, i, 0)),
            pl.BlockSpec((1, BT, 1), lambda i: (0, i, 0)),
            pl.BlockSpec((1, BT, 1), lambda i: (1, i, 0)),
            pl.BlockSpec((BT, D), lambda i: (i, 0)),
            pl.BlockSpec((D, H), lambda i: (0, 0)),
            pl.BlockSpec((D, H), lambda i: (0, 0)),
            pl.BlockSpec((H, OUT), lambda i: (0, 0)),
            pl.BlockSpec((1, H), lambda i: (0, 0)),
            pl.BlockSpec((1, OUT), lambda i: (0, 0)),
        ],
        out_specs=pl.BlockSpec((BT, OUT), lambda i: (i, 0)),
        out_shape=jax.ShapeDtypeStruct((N_EXP, OUT), jnp.float32),
    )(parts, parts, cnts, cnts, x_expert, wlT, wrT, woT, bl, bo)


def kernel(x_loc, x_expert, edge_index, W_l, b_l, W_r, W_lin, b_lin):
    src = edge_index[0]
    dst = edge_index[1]
    pad = IDX_ROWS * K - E
    src_p = jnp.concatenate([src, jnp.zeros((pad,), jnp.int32)])
    dst_p = jnp.concatenate([dst, jnp.full((pad,), N_EXP, jnp.int32)])
    idx2d = jnp.concatenate([src_p.reshape(IDX_ROWS, K),
                             dst_p.reshape(IDX_ROWS, K)], axis=1)
    zrows = jnp.zeros((N_ACC, D), jnp.float32)

    parts, cnts, _ = _sc_segment_sum(x_loc, idx2d, zrows)
    return _tc_stage(parts, cnts.reshape(NC, N_ACC, 1), x_expert,
                     W_l.T, W_r.T, W_lin.T, b_l[None, :], b_lin[None, :])


# speedup vs baseline: 1.0142x; 1.0089x over previous
import functools

import jax
import jax.numpy as jnp
from jax import lax
from jax.experimental import pallas as pl
from jax.experimental.pallas import tpu as pltpu
from jax.experimental.pallas import tpu_sc as plsc

N_LOC = 10000
N_EXP = 10000
E = 320000
D = 128
H = 128
OUT = 128

NC = 2
NS = 16
NW = NC * NS
K = 128
NB = 2
GA = 1
NI = 8
NCH = 80
EPW = NCH * K
E_PAD = NW * EPW
IDX_ROWS = NW * NCH + NI
N_ACC = 10240
RPS = N_ACC // NS
G16 = K // 16

_sc_mesh = plsc.VectorSubcoreMesh(core_axis_name="c", subcore_axis_name="s")


@functools.partial(
    pl.kernel,
    mesh=_sc_mesh,
    compiler_params=pltpu.CompilerParams(needs_layout_passes=False),
    out_type=(
        jax.ShapeDtypeStruct((NC, N_ACC, D), jnp.float32),
        jax.ShapeDtypeStruct((NC, N_ACC), jnp.float32),
        jax.ShapeDtypeStruct((NC, NS, N_ACC), jnp.float32),
    ),
    scratch_types=[
        [pltpu.VMEM((2 * K,), jnp.int32) for _ in range(NI)],
        [pltpu.VMEM((K, D), jnp.float32) for _ in range(NB)],
        [pltpu.VMEM((K,), jnp.int32) for _ in range(NB)],
        pltpu.VMEM((N_ACC,), jnp.float32),
        pltpu.VMEM((RPS,), jnp.float32),
        pltpu.VMEM((RPS,), jnp.float32),
        pltpu.VMEM_SHARED((N_ACC, D), jnp.float32),
        [pltpu.SemaphoreType.DMA for _ in range(NI)],
        [pltpu.SemaphoreType.DMA for _ in range(NB)],
        [pltpu.SemaphoreType.DMA for _ in range(NB)],
    ],
)
def _sc_segment_sum(x_loc, idx2d, zrows, out_feat, out_cnt, out_hist,
                    islots, bufs, dsts, hist_v, cred_v, tmp_v,
                    acc_sh, isems, gsems, ssems):
    c = lax.axis_index("c")
    s = lax.axis_index("s")
    wid = s * NC + c
    row0 = wid * NCH

    def idx_issue(m, slot):
        pltpu.async_copy(idx2d.at[row0 + m], islots[slot], isems[slot])

    def idx_wait(m, slot):
        pltpu.make_async_copy(idx2d.at[row0 + m], islots[slot],
                              isems[slot]).wait()

    def gather_issue(slot, b):
        pltpu.async_copy(x_loc.at[islots[slot].at[pl.ds(0, K)]], bufs[b],
                         gsems[b])

    def gather_wait(slot, b):
        pltpu.make_async_copy(x_loc.at[islots[slot].at[pl.ds(0, K)]], bufs[b],
                              gsems[b]).wait()

    def scatter_issue(b):
        pltpu.async_copy(bufs[b], acc_sh.at[dsts[b]], ssems[b], add=True)

    def scatter_wait(b):
        pltpu.make_async_copy(bufs[b], acc_sh.at[dsts[b]], ssems[b]).wait()

    zeros16 = jnp.zeros((16,), jnp.float32)

    def zh(k, carry):
        hist_v[pl.ds(k * 16, 16)] = zeros16
        return carry

    lax.fori_loop(0, N_ACC // 16, zh, 0)
    pltpu.sync_copy(zrows.at[pl.ds(s * RPS, RPS)],
                    acc_sh.at[pl.ds(s * RPS, RPS)])
    plsc.subcore_barrier()

    ones16 = jnp.ones((16,), jnp.float32)

    for m in range(NI - 1):
        idx_issue(m, m)
    for m in range(GA):
        idx_wait(m, m)
        gather_issue(m, m)

    def outer(i, carry):
        for b in range(NI):
            ch = i * NI + b
            idx_issue(ch + NI - 1, (b + NI - 1) % NI)
            if b < NB - GA:
                @pl.when(i >= 1)
                def _drain():
                    scatter_wait((b + GA) % NB)
            else:
                scatter_wait((b + GA) % NB)
            idx_wait(ch + GA, (b + GA) % NI)
            gather_issue((b + GA) % NI, (b + GA) % NB)
            gather_wait(b % NI, b % NB)
            for g in range(G16):
                v = islots[b % NI][pl.ds(K + g * 16, 16)]
                dsts[b % NB][pl.ds(g * 16, 16)] = v
                plsc.addupdate_scatter(hist_v, [v], ones16)
            scatter_issue(b % NB)
        return carry

    lax.fori_loop(0, NCH // NI, outer, 0)

    for m in range(GA):
        gather_wait((NCH + m) % NI, (NCH + m) % NB)
    for m in range(GA, NI - 1):
        idx_wait(NCH + m, (NCH + m) % NI)
    for m in range(NB - GA):
        scatter_wait((NCH - (NB - GA) + m) % NB)

    pltpu.sync_copy(hist_v, out_hist.at[c].at[s])
    plsc.subcore_barrier()

    def czero(g, carry):
        cred_v[pl.ds(g * 16, 16)] = zeros16
        return carry

    lax.fori_loop(0, RPS // 16, czero, 0)
    for j in range(NS):
        pltpu.sync_copy(out_hist.at[c].at[j].at[pl.ds(s * RPS, RPS)], tmp_v)

        def cadd(g, carry):
            sl = pl.ds(g * 16, 16)
            cred_v[sl] = cred_v[sl] + tmp_v[sl]
            return carry

        lax.fori_loop(0, RPS // 16, cadd, 0)

    pltpu.sync_copy(acc_sh.at[pl.ds(s * RPS, RPS)],
                    out_feat.at[c].at[pl.ds(s * RPS, RPS)])
    pltpu.sync_copy(cred_v, out_cnt.at[c].at[pl.ds(s * RPS, RPS)])


def _tc_body(p0, p1, c0, c1, xe, wl, wr, wo, bl, bo, o):
    cnt = jnp.maximum(c0[0] + c1[0], 1.0)
    sacc = p0[0] + p1[0]
    mean = sacc / cnt
    h = jnp.dot(mean, wl[...], preferred_element_type=jnp.float32)
    h = h + jnp.dot(xe[...], wr[...], preferred_element_type=jnp.float32)
    h = jnp.maximum(h + bl[...], 0.0)
    o[...] = jnp.dot(h, wo[...], preferred_element_type=jnp.float32) + bo[...]


BT = 1024


def _tc_stage(parts, cnts, x_expert, wlT, wrT, woT, bl, bo):
    grid = (-(-N_EXP // BT),)
    return pl.pallas_call(
        _tc_body,
        grid=grid,
        in_specs=[
            pl.BlockSpec((1, BT, D), lambda i: (0, i, 0)),
            pl.BlockSpec((1, BT, D), lambda i: (1, i, 0)),
            pl.BlockSpec((1, BT, 1), lambda i: (0, i, 0)),
            pl.BlockSpec((1, BT, 1), lambda i: (1, i, 0)),
            pl.BlockSpec((BT, D), lambda i: (i, 0)),
            pl.BlockSpec((D, H), lambda i: (0, 0)),
            pl.BlockSpec((D, H), lambda i: (0, 0)),
            pl.BlockSpec((H, OUT), lambda i: (0, 0)),
            pl.BlockSpec((1, H), lambda i: (0, 0)),
            pl.BlockSpec((1, OUT), lambda i: (0, 0)),
        ],
        out_specs=pl.BlockSpec((BT, OUT), lambda i: (i, 0)),
        out_shape=jax.ShapeDtypeStruct((N_EXP, OUT), jnp.float32),
    )(parts, parts, cnts, cnts, x_expert, wlT, wrT, woT, bl, bo)


def kernel(x_loc, x_expert, edge_index, W_l, b_l, W_r, W_lin, b_lin):
    src = edge_index[0]
    dst = edge_index[1]
    pad = IDX_ROWS * K - E
    src_p = jnp.concatenate([src, jnp.zeros((pad,), jnp.int32)])
    dst_p = jnp.concatenate([dst, jnp.full((pad,), N_EXP, jnp.int32)])
    idx2d = jnp.concatenate([src_p.reshape(IDX_ROWS, K),
                             dst_p.reshape(IDX_ROWS, K)], axis=1)
    zrows = jnp.zeros((N_ACC, D), jnp.float32)

    parts, cnts, _ = _sc_segment_sum(x_loc, idx2d, zrows)
    return _tc_stage(parts, cnts.reshape(NC, N_ACC, 1), x_expert,
                     W_l.T, W_r.T, W_lin.T, b_l[None, :], b_lin[None, :])


# speedup vs baseline: 1.0147x; 1.0005x over previous
import functools

import jax
import jax.numpy as jnp
from jax import lax
from jax.experimental import pallas as pl
from jax.experimental.pallas import tpu as pltpu
from jax.experimental.pallas import tpu_sc as plsc

N_LOC = 10000
N_EXP = 10000
E = 320000
D = 128
H = 128
OUT = 128

NC = 2
NS = 16
NW = NC * NS
K = 128
NB = 2
GA = 1
NI = 8
NCH = 80
EPW = NCH * K
E_PAD = NW * EPW
IDX_ROWS = NW * NCH + NI
N_ACC = 10240
RPS = N_ACC // NS
G16 = K // 16

_sc_mesh = plsc.VectorSubcoreMesh(core_axis_name="c", subcore_axis_name="s")


@functools.partial(
    pl.kernel,
    mesh=_sc_mesh,
    compiler_params=pltpu.CompilerParams(needs_layout_passes=False),
    out_type=(
        jax.ShapeDtypeStruct((NC, N_ACC, D), jnp.float32),
        jax.ShapeDtypeStruct((NC, N_ACC), jnp.float32),
        jax.ShapeDtypeStruct((NC, NS, N_ACC), jnp.float32),
    ),
    scratch_types=[
        [pltpu.VMEM((2 * K,), jnp.int32) for _ in range(NI)],
        [pltpu.VMEM((K, D), jnp.float32) for _ in range(NB)],
        [pltpu.VMEM((K,), jnp.int32) for _ in range(NB)],
        pltpu.VMEM((N_ACC,), jnp.float32),
        pltpu.VMEM((RPS,), jnp.float32),
        pltpu.VMEM((RPS,), jnp.float32),
        pltpu.VMEM_SHARED((N_ACC, D), jnp.float32),
        [pltpu.SemaphoreType.DMA for _ in range(NI)],
        [pltpu.SemaphoreType.DMA for _ in range(NB)],
        [pltpu.SemaphoreType.DMA for _ in range(NB)],
    ],
)
def _sc_segment_sum(x_loc, idx2d, zrows, out_feat, out_cnt, out_hist,
                    islots, bufs, dsts, hist_v, cred_v, tmp_v,
                    acc_sh, isems, gsems, ssems):
    c = lax.axis_index("c")
    s = lax.axis_index("s")
    wid = s * NC + c
    row0 = wid * NCH

    def idx_issue(m, slot):
        pltpu.async_copy(idx2d.at[row0 + m], islots[slot], isems[slot])

    def idx_wait(m, slot):
        pltpu.make_async_copy(idx2d.at[row0 + m], islots[slot],
                              isems[slot]).wait()

    def gather_issue(slot, b):
        pltpu.async_copy(x_loc.at[islots[slot].at[pl.ds(0, K)]], bufs[b],
                         gsems[b])

    def gather_wait(slot, b):
        pltpu.make_async_copy(x_loc.at[islots[slot].at[pl.ds(0, K)]], bufs[b],
                              gsems[b]).wait()

    def scatter_issue(b):
        pltpu.async_copy(bufs[b], acc_sh.at[dsts[b]], ssems[b], add=True)

    def scatter_wait(b):
        pltpu.make_async_copy(bufs[b], acc_sh.at[dsts[b]], ssems[b]).wait()

    zeros16 = jnp.zeros((16,), jnp.float32)

    def zh(k, carry):
        hist_v[pl.ds(k * 16, 16)] = zeros16
        return carry

    lax.fori_loop(0, N_ACC // 16, zh, 0)
    pltpu.sync_copy(zrows.at[pl.ds(s * RPS, RPS)],
                    acc_sh.at[pl.ds(s * RPS, RPS)])
    plsc.subcore_barrier()

    ones16 = jnp.ones((16,), jnp.float32)

    for m in range(NI - 1):
        idx_issue(m, m)
    for m in range(GA):
        idx_wait(m, m)
        gather_issue(m, m)

    def outer(i, carry):
        for b in range(NI):
            ch = i * NI + b
            idx_issue(ch + NI - 1, (b + NI - 1) % NI)
            if b < NB - GA:
                @pl.when(i >= 1)
                def _drain():
                    scatter_wait((b + GA) % NB)
            else:
                scatter_wait((b + GA) % NB)
            idx_wait(ch + GA, (b + GA) % NI)
            gather_issue((b + GA) % NI, (b + GA) % NB)
            gather_wait(b % NI, b % NB)
            for g in range(G16):
                v = islots[b % NI][pl.ds(K + g * 16, 16)]
                dsts[b % NB][pl.ds(g * 16, 16)] = v
                plsc.addupdate_scatter(hist_v, [v], ones16)
            scatter_issue(b % NB)
        return carry

    lax.fori_loop(0, NCH // NI, outer, 0)

    for m in range(GA):
        gather_wait((NCH + m) % NI, (NCH + m) % NB)
    for m in range(GA, NI - 1):
        idx_wait(NCH + m, (NCH + m) % NI)
    for m in range(NB - GA):
        scatter_wait((NCH - (NB - GA) + m) % NB)

    pltpu.sync_copy(hist_v, out_hist.at[c].at[s])
    plsc.subcore_barrier()

    def czero(g, carry):
        cred_v[pl.ds(g * 16, 16)] = zeros16
        return carry

    lax.fori_loop(0, RPS // 16, czero, 0)
    for j in range(NS):
        pltpu.sync_copy(out_hist.at[c].at[j].at[pl.ds(s * RPS, RPS)], tmp_v)

        def cadd(g, carry):
            sl = pl.ds(g * 16, 16)
            cred_v[sl] = cred_v[sl] + tmp_v[sl]
            return carry

        lax.fori_loop(0, RPS // 16, cadd, 0)

    pltpu.sync_copy(acc_sh.at[pl.ds(s * RPS, RPS)],
                    out_feat.at[c].at[pl.ds(s * RPS, RPS)])
    pltpu.sync_copy(cred_v, out_cnt.at[c].at[pl.ds(s * RPS, RPS)])


def _tc_pre_body(xe, wr, bl, o):
    o[...] = jnp.dot(xe[...], wr[...],
                     preferred_element_type=jnp.float32) + bl[...]


def _tc_body(p0, p1, c0, c1, pre, wl, wo, bo, o):
    cnt = jnp.maximum(c0[0] + c1[0], 1.0)
    sacc = p0[0] + p1[0]
    mean = sacc / cnt
    h = jnp.dot(mean, wl[...], preferred_element_type=jnp.float32)
    h = jnp.maximum(h + pre[...], 0.0)
    o[...] = jnp.dot(h, wo[...], preferred_element_type=jnp.float32) + bo[...]


BT = 1024


def _tc_pre_stage(x_expert, wrT, bl):
    grid = (-(-N_EXP // BT),)
    return pl.pallas_call(
        _tc_pre_body,
        grid=grid,
        in_specs=[
            pl.BlockSpec((BT, D), lambda i: (i, 0)),
            pl.BlockSpec((D, H), lambda i: (0, 0)),
            pl.BlockSpec((1, H), lambda i: (0, 0)),
        ],
        out_specs=pl.BlockSpec((BT, H), lambda i: (i, 0)),
        out_shape=jax.ShapeDtypeStruct((N_EXP, H), jnp.float32),
    )(x_expert, wrT, bl)


def _tc_stage(parts, cnts, pre, wlT, woT, bo):
    grid = (-(-N_EXP // BT),)
    return pl.pallas_call(
        _tc_body,
        grid=grid,
        in_specs=[
            pl.BlockSpec((1, BT, D), lambda i: (0, i, 0)),
            pl.BlockSpec((1, BT, D), lambda i: (1, i, 0)),
            pl.BlockSpec((1, BT, 1), lambda i: (0, i, 0)),
            pl.BlockSpec((1, BT, 1), lambda i: (1, i, 0)),
            pl.BlockSpec((BT, H), lambda i: (i, 0)),
            pl.BlockSpec((D, H), lambda i: (0, 0)),
            pl.BlockSpec((H, OUT), lambda i: (0, 0)),
            pl.BlockSpec((1, OUT), lambda i: (0, 0)),
        ],
        out_specs=pl.BlockSpec((BT, OUT), lambda i: (i, 0)),
        out_shape=jax.ShapeDtypeStruct((N_EXP, OUT), jnp.float32),
    )(parts, parts, cnts, cnts, pre, wlT, woT, bo)


def kernel(x_loc, x_expert, edge_index, W_l, b_l, W_r, W_lin, b_lin):
    src = edge_index[0]
    dst = edge_index[1]
    pad = IDX_ROWS * K - E
    src_p = jnp.concatenate([src, jnp.zeros((pad,), jnp.int32)])
    dst_p = jnp.concatenate([dst, jnp.full((pad,), N_EXP, jnp.int32)])
    idx2d = jnp.concatenate([src_p.reshape(IDX_ROWS, K),
                             dst_p.reshape(IDX_ROWS, K)], axis=1)
    zrows = jnp.zeros((N_ACC, D), jnp.float32)

    pre = _tc_pre_stage(x_expert, W_r.T, b_l[None, :])
    parts, cnts, _ = _sc_segment_sum(x_loc, idx2d, zrows)
    return _tc_stage(parts, cnts.reshape(NC, N_ACC, 1), pre,
                     W_l.T, W_lin.T, b_lin[None, :])
